# baseline (speedup 1.0000x reference)
import jax
import jax.numpy as jnp
from jax.experimental import pallas as pl
from jax.experimental.pallas import tpu as pltpu

D_MODEL = 768
N_EXP = 8
EXP_DIM = 3072
SEQ = 2048
TS = 512
HALF = TS // 2
NS = SEQ // TS
UR = D_MODEL // NS
DR = EXP_DIM // NS


def _gelu_bf16(h):
    k0 = jnp.bfloat16(0.7978845608028654)
    k1 = jnp.bfloat16(0.7978845608028654 * 0.044715)
    t = jnp.tanh(h * (k0 + k1 * h * h))
    return (jnp.bfloat16(0.5) * h) * (jnp.bfloat16(1.0) + t)


def _moe_kernel(x_ref, rw_ref, rb_ref, wup_hbm, bup_ref, wdn_hbm, bdn_ref,
                out_ref, probs_ref, wup0, wdn0, wup1, wdn1, land_up, land_dn,
                sem_up, sem_dn):
    e = pl.program_id(0)
    s = pl.program_id(1)
    first = e == 0
    nxt = jnp.minimum(e + 1, N_EXP - 1)
    nxt2 = jnp.minimum(e + 2, N_EXP - 1)

    def _copy_pair(exp, chunk, slot):
        up = pltpu.make_async_copy(
            wup_hbm.at[exp, pl.ds(chunk * UR, UR), :],
            land_up.at[slot], sem_up.at[slot])
        dn = pltpu.make_async_copy(
            wdn_hbm.at[exp, pl.ds(chunk * DR, DR), :],
            land_dn.at[slot], sem_dn.at[slot])
        return up, dn

    @pl.when(jnp.logical_and(first, s == 0))
    def _():
        def body(c, carry):
            up, dn = _copy_pair(0, c, 0)
            up.start()
            dn.start()
            up2, dn2 = _copy_pair(0, c, 0)
            up2.wait()
            dn2.wait()
            wup0[pl.ds(c * UR, UR), :] = land_up[0].astype(jnp.bfloat16)
            wdn0[pl.ds(c * DR, DR), :] = land_dn[0].astype(jnp.bfloat16)
            return carry
        jax.lax.fori_loop(0, NS, body, 0)
        up, dn = _copy_pair(1, 0, 0)
        up.start()
        dn.start()

    up_w, dn_w = _copy_pair(nxt, s, jax.lax.rem(s, 2))
    up_w.wait()
    dn_w.wait()

    e_start = jnp.where(s < NS - 1, nxt, nxt2)
    c_start = jax.lax.rem(s + 1, NS)
    up_s, dn_s = _copy_pair(e_start, c_start, jax.lax.rem(s + 1, 2))
    up_s.start()
    dn_s.start()

    xs_bf = x_ref[...].astype(jnp.bfloat16)

    @pl.when(first)
    def _():
        logits = jnp.dot(xs_bf, rw_ref[...].astype(jnp.bfloat16),
                         preferred_element_type=jnp.float32) + rb_ref[...]
        m = jnp.max(logits, axis=-1, keepdims=True)
        ex = jnp.exp(logits - m)
        probs_ref[pl.ds(s * TS, TS), :] = ex / jnp.sum(ex, axis=-1,
                                                       keepdims=True)

    probs = probs_ref[pl.ds(s * TS, TS), :]
    lane = jax.lax.broadcasted_iota(jnp.int32, (TS, N_EXP), 1)
    p_e = jnp.sum(jnp.where(lane == e, probs, 0.0), axis=1, keepdims=True)
    bup = bup_ref[0, 0].astype(jnp.bfloat16)[None, :]
    slot = jax.lax.rem(s, 2)

    def _step(wu_rd, wd_rd, wu_wr, wd_wr):
        wu_wr[pl.ds(s * UR, UR), :] = land_up[slot].astype(jnp.bfloat16)
        wd_wr[pl.ds(s * DR, DR), :] = land_dn[slot].astype(jnp.bfloat16)

        h32_a = jnp.dot(xs_bf[:HALF], wu_rd[...],
                        preferred_element_type=jnp.float32)
        h32_b = jnp.dot(xs_bf[HALF:], wu_rd[...],
                        preferred_element_type=jnp.float32)
        g_a = _gelu_bf16(h32_a.astype(jnp.bfloat16) + bup)
        g_b = _gelu_bf16(h32_b.astype(jnp.bfloat16) + bup)
        c_a = jnp.dot(g_a, wd_rd[...], preferred_element_type=jnp.float32)
        c_b = jnp.dot(g_b, wd_rd[...], preferred_element_type=jnp.float32)
        contrib = jnp.concatenate([c_a, c_b], axis=0)

        bias_term = jnp.dot(probs.astype(jnp.bfloat16),
                            bdn_ref[...].astype(jnp.bfloat16),
                            preferred_element_type=jnp.float32)
        base = jnp.where(first, bias_term, out_ref[pl.ds(s * TS, TS), :])
        out_ref[pl.ds(s * TS, TS), :] = base + contrib * p_e

    @pl.when(jax.lax.rem(e, 2) == 0)
    def _():
        _step(wup0, wdn0, wup1, wdn1)

    @pl.when(jax.lax.rem(e, 2) == 1)
    def _():
        _step(wup1, wdn1, wup0, wdn0)

    @pl.when(jnp.logical_and(e == N_EXP - 1, s == NS - 1))
    def _():
        up, dn = _copy_pair(nxt2, 0, jax.lax.rem(NS, 2))
        up.wait()
        dn.wait()


@jax.jit
def _moe(x2, router_w, router_b, w_up, b_up3, w_down, b_down):
    grid = (N_EXP, NS)
    return pl.pallas_call(
        _moe_kernel,
        grid=grid,
        in_specs=[
            pl.BlockSpec((TS, D_MODEL), lambda e, s: (s, 0)),
            pl.BlockSpec((D_MODEL, N_EXP), lambda e, s: (0, 0)),
            pl.BlockSpec((N_EXP,), lambda e, s: (0,)),
            pl.BlockSpec(memory_space=pltpu.MemorySpace.HBM),
            pl.BlockSpec((1, 1, EXP_DIM), lambda e, s: (e, 0, 0)),
            pl.BlockSpec(memory_space=pltpu.MemorySpace.HBM),
            pl.BlockSpec((N_EXP, D_MODEL), lambda e, s: (0, 0)),
        ],
        out_specs=pl.BlockSpec((SEQ, D_MODEL), lambda e, s: (0, 0)),
        out_shape=jax.ShapeDtypeStruct((SEQ, D_MODEL), jnp.float32),
        scratch_shapes=[
            pltpu.VMEM((SEQ, N_EXP), jnp.float32),
            pltpu.VMEM((D_MODEL, EXP_DIM), jnp.bfloat16),
            pltpu.VMEM((EXP_DIM, D_MODEL), jnp.bfloat16),
            pltpu.VMEM((D_MODEL, EXP_DIM), jnp.bfloat16),
            pltpu.VMEM((EXP_DIM, D_MODEL), jnp.bfloat16),
            pltpu.VMEM((2, UR, EXP_DIM), jnp.float32),
            pltpu.VMEM((2, DR, D_MODEL), jnp.float32),
            pltpu.SemaphoreType.DMA((2,)),
            pltpu.SemaphoreType.DMA((2,)),
        ],
        compiler_params=pltpu.CompilerParams(
            dimension_semantics=("arbitrary", "arbitrary"),
        ),
    )(x2, router_w, router_b, w_up, b_up3, w_down, b_down)


def kernel(x, router_w, router_b, w_up, b_up, w_down, b_down):
    b, seq, d = x.shape
    out = _moe(x.reshape(seq, d), router_w, router_b, w_up,
               b_up.reshape(N_EXP, 1, EXP_DIM), w_down, b_down)
    return out.reshape(b, seq, d)


# speedup vs baseline: 1.0325x; 1.0325x over previous
import jax
import jax.numpy as jnp
from jax.experimental import pallas as pl
from jax.experimental.pallas import tpu as pltpu

D_MODEL = 768
N_EXP = 8
EXP_DIM = 3072
SEQ = 2048
TS = 512
HALF = TS // 2
NS = SEQ // TS
UR = D_MODEL // NS
DR = EXP_DIM // NS


def _gelu_bf16(h):
    k0 = jnp.bfloat16(0.7978845608028654)
    k1 = jnp.bfloat16(0.7978845608028654 * 0.044715)
    t = jnp.tanh(h * (k0 + k1 * h * h))
    return (jnp.bfloat16(0.5) * h) * (jnp.bfloat16(1.0) + t)


def _moe_kernel(x_ref, rw_ref, rb_ref, wup_hbm, bup_ref, wdn_hbm, bdn_ref,
                out_ref, probs_ref, wup_bf, wdn_bf, land_up, land_dn,
                sem_up, sem_dn):
    e = pl.program_id(0)
    s = pl.program_id(1)
    first = e == 0
    nxt = e + 1
    bank_use = jax.lax.rem(e, 2)
    bank_nxt = jax.lax.rem(nxt, 2)

    def _start(chunk, slot):
        pltpu.make_async_copy(
            wup_hbm.at[nxt, pl.ds(chunk * UR, UR), :],
            land_up.at[slot], sem_up.at[slot]).start()
        pltpu.make_async_copy(
            wdn_hbm.at[nxt, pl.ds(chunk * DR, DR), :],
            land_dn.at[slot], sem_dn.at[slot]).start()

    def _finish(chunk, slot, bank):
        pltpu.make_async_copy(
            wup_hbm.at[nxt, pl.ds(chunk * UR, UR), :],
            land_up.at[slot], sem_up.at[slot]).wait()
        pltpu.make_async_copy(
            wdn_hbm.at[nxt, pl.ds(chunk * DR, DR), :],
            land_dn.at[slot], sem_dn.at[slot]).wait()
        wup_bf[bank, pl.ds(chunk * UR, UR), :] = land_up[slot].astype(
            jnp.bfloat16)
        wdn_bf[bank, pl.ds(chunk * DR, DR), :] = land_dn[slot].astype(
            jnp.bfloat16)

    @pl.when(jnp.logical_and(first, s == 0))
    def _():
        def body(c, carry):
            pltpu.make_async_copy(
                wup_hbm.at[0, pl.ds(c * UR, UR), :],
                land_up.at[0], sem_up.at[0]).start()
            pltpu.make_async_copy(
                wdn_hbm.at[0, pl.ds(c * DR, DR), :],
                land_dn.at[0], sem_dn.at[0]).start()
            pltpu.make_async_copy(
                wup_hbm.at[0, pl.ds(c * UR, UR), :],
                land_up.at[0], sem_up.at[0]).wait()
            pltpu.make_async_copy(
                wdn_hbm.at[0, pl.ds(c * DR, DR), :],
                land_dn.at[0], sem_dn.at[0]).wait()
            wup_bf[0, pl.ds(c * UR, UR), :] = land_up[0].astype(jnp.bfloat16)
            wdn_bf[0, pl.ds(c * DR, DR), :] = land_dn[0].astype(jnp.bfloat16)
            return carry
        jax.lax.fori_loop(0, NS, body, 0)

    @pl.when(jnp.logical_and(s == 0, e >= 1))
    def _():
        _finish(NS - 1, (NS - 1) % 2, bank_use)

    @pl.when(e < N_EXP - 1)
    def _():
        _start(s, jax.lax.rem(s, 2))

    @pl.when(jnp.logical_and(e < N_EXP - 1, s >= 1))
    def _():
        _finish(s - 1, jax.lax.rem(s - 1, 2), bank_nxt)

    xs_bf = x_ref[...].astype(jnp.bfloat16)

    @pl.when(first)
    def _():
        logits = jnp.dot(xs_bf, rw_ref[...].astype(jnp.bfloat16),
                         preferred_element_type=jnp.float32) + rb_ref[...]
        m = jnp.max(logits, axis=-1, keepdims=True)
        ex = jnp.exp(logits - m)
        probs_ref[pl.ds(s * TS, TS), :] = ex / jnp.sum(ex, axis=-1,
                                                       keepdims=True)

    probs = probs_ref[pl.ds(s * TS, TS), :]
    lane = jax.lax.broadcasted_iota(jnp.int32, (TS, N_EXP), 1)
    p_e = jnp.sum(jnp.where(lane == e, probs, 0.0), axis=1, keepdims=True)

    wu = wup_bf[bank_use]
    wd = wdn_bf[bank_use]
    bup = bup_ref[0, 0].astype(jnp.bfloat16)[None, :]

    h32_a = jnp.dot(xs_bf[:HALF], wu, preferred_element_type=jnp.float32)
    h32_b = jnp.dot(xs_bf[HALF:], wu, preferred_element_type=jnp.float32)
    g_a = _gelu_bf16(h32_a.astype(jnp.bfloat16) + bup)
    g_b = _gelu_bf16(h32_b.astype(jnp.bfloat16) + bup)
    c_a = jnp.dot(g_a, wd, preferred_element_type=jnp.float32)
    c_b = jnp.dot(g_b, wd, preferred_element_type=jnp.float32)
    contrib = jnp.concatenate([c_a, c_b], axis=0)

    @pl.when(first)
    def _():
        out_ref[pl.ds(s * TS, TS), :] = contrib * p_e + jnp.dot(
            probs.astype(jnp.bfloat16), bdn_ref[...].astype(jnp.bfloat16),
            preferred_element_type=jnp.float32)

    @pl.when(jnp.logical_not(first))
    def _():
        out_ref[pl.ds(s * TS, TS), :] += contrib * p_e


@jax.jit
def _moe(x2, router_w, router_b, w_up, b_up3, w_down, b_down):
    grid = (N_EXP, NS)
    return pl.pallas_call(
        _moe_kernel,
        grid=grid,
        in_specs=[
            pl.BlockSpec((TS, D_MODEL), lambda e, s: (s, 0)),
            pl.BlockSpec((D_MODEL, N_EXP), lambda e, s: (0, 0)),
            pl.BlockSpec((N_EXP,), lambda e, s: (0,)),
            pl.BlockSpec(memory_space=pltpu.MemorySpace.HBM),
            pl.BlockSpec((1, 1, EXP_DIM), lambda e, s: (e, 0, 0)),
            pl.BlockSpec(memory_space=pltpu.MemorySpace.HBM),
            pl.BlockSpec((N_EXP, D_MODEL), lambda e, s: (0, 0)),
        ],
        out_specs=pl.BlockSpec((SEQ, D_MODEL), lambda e, s: (0, 0)),
        out_shape=jax.ShapeDtypeStruct((SEQ, D_MODEL), jnp.float32),
        scratch_shapes=[
            pltpu.VMEM((SEQ, N_EXP), jnp.float32),
            pltpu.VMEM((2, D_MODEL, EXP_DIM), jnp.bfloat16),
            pltpu.VMEM((2, EXP_DIM, D_MODEL), jnp.bfloat16),
            pltpu.VMEM((2, UR, EXP_DIM), jnp.float32),
            pltpu.VMEM((2, DR, D_MODEL), jnp.float32),
            pltpu.SemaphoreType.DMA((2,)),
            pltpu.SemaphoreType.DMA((2,)),
        ],
        compiler_params=pltpu.CompilerParams(
            dimension_semantics=("arbitrary", "arbitrary"),
        ),
    )(x2, router_w, router_b, w_up, b_up3, w_down, b_down)


def kernel(x, router_w, router_b, w_up, b_up, w_down, b_down):
    b, seq, d = x.shape
    out = _moe(x.reshape(seq, d), router_w, router_b, w_up,
               b_up.reshape(N_EXP, 1, EXP_DIM), w_down, b_down)
    return out.reshape(b, seq, d)
